# baseline (speedup 1.0000x reference)
import jax
import jax.numpy as jnp
from jax.experimental import pallas as pl
from jax.experimental.pallas import tpu as pltpu


def _tower_kernel(x1_ref, w1_ref, b1_ref, w2_ref, b2_ref, o_ref, p1_scr,
                  vp2_scr):
    B = x1_ref.shape[0]
    x1 = x1_ref[...].reshape(B * 200, 36)
    c1 = jnp.dot(x1, w1_ref[...], preferred_element_type=jnp.float32)
    c1 = c1.reshape(B, 200, 128)
    hp = jnp.maximum(jnp.maximum(c1[:, :, 0:32], c1[:, :, 32:64]),
                     jnp.maximum(c1[:, :, 64:96], c1[:, :, 96:128]))
    p1 = jnp.maximum(hp + b1_ref[...], 0.0)

    p1_scr[...] = jnp.zeros_like(p1_scr)
    for r in range(14):
        p1_scr[:, pl.ds((r + 1) * 16 + 1, 14), :] = p1[:, r * 14:(r + 1) * 14, :]

    p1f = p1_scr[...].astype(jnp.bfloat16)
    patch = jnp.concatenate(
        [p1f[:, dy * 16 + dx:dy * 16 + dx + 224, :]
         for dy in range(3) for dx in range(3)], axis=-1)
    c2 = jnp.dot(patch.reshape(B * 224, 288), w2_ref[...],
                 preferred_element_type=jnp.float32)
    c2 = c2.reshape(B, 7, 32, 64)
    vp2_scr[...] = jnp.maximum(c2[:, :, 0:14], c2[:, :, 16:30])
    hp2 = jnp.maximum(vp2_scr[:, :, pl.ds(0, 7, stride=2), :],
                      vp2_scr[:, :, pl.ds(1, 7, stride=2), :])
    o_ref[...] = jnp.maximum(hp2 + b2_ref[...], 0.0).astype(o_ref.dtype)


def _mlp_kernel(x_ref, w1_ref, b1_ref, w2_ref, b2_ref, o_ref):
    h = jnp.dot(x_ref[...], w1_ref[...], preferred_element_type=jnp.float32)
    h = jnp.maximum(h + b1_ref[...], 0.0).astype(jnp.bfloat16)
    o_ref[...] = jnp.dot(h, w2_ref[...],
                         preferred_element_type=jnp.float32) + b2_ref[...]


def _pick_block(n, pref):
    b = pref
    while n % b:
        b //= 2
    return max(b, 1)


def kernel(x_nchw, w1_hwio, b1, w2_hwio, b2, fw1, fb1, fw2, fb2):
    N = x_nchw.shape[0]
    x = x_nchw[:, 0]
    xp = jnp.pad(x, ((0, 0), (1, 1), (1, 1))).astype(jnp.bfloat16)

    x1 = jnp.stack([xp[:, s + dy:s + dy + 28:2, u + dx:u + dx + 28:2]
                    for s in range(2) for u in range(2)
                    for dy in range(3) for dx in range(3)], axis=-1)
    x1 = jnp.pad(x1.reshape(N, 196, 36), ((0, 0), (0, 4), (0, 0)))

    w1f = w1_hwio.reshape(9, 32)
    w1p = (jnp.eye(4, dtype=jnp.float32)[:, None, :, None]
           * w1f[None, :, None, :]).reshape(36, 128).astype(jnp.bfloat16)
    w2p = w2_hwio.reshape(9 * 32, 64).astype(jnp.bfloat16)

    bb = _pick_block(N, 32)
    feat = pl.pallas_call(
        _tower_kernel,
        out_shape=jax.ShapeDtypeStruct((N, 7, 7, 64), jnp.bfloat16),
        grid=(N // bb,),
        in_specs=[
            pl.BlockSpec((bb, 200, 36), lambda n: (n, 0, 0)),
            pl.BlockSpec((36, 128), lambda n: (0, 0)),
            pl.BlockSpec((1, 32), lambda n: (0, 0)),
            pl.BlockSpec((288, 64), lambda n: (0, 0)),
            pl.BlockSpec((1, 64), lambda n: (0, 0)),
        ],
        out_specs=pl.BlockSpec((bb, 7, 7, 64), lambda n: (n, 0, 0, 0)),
        scratch_shapes=[pltpu.VMEM((bb, 272, 32), jnp.float32),
                        pltpu.VMEM((bb, 7, 14, 64), jnp.float32)],
        compiler_params=pltpu.CompilerParams(dimension_semantics=("parallel",)),
    )(x1, w1p, b1.reshape(1, 32), w2p, b2.reshape(1, 64))

    feat = feat.reshape(N, 3136)

    w1t = fw1.reshape(128, 64, 7, 7).transpose(2, 3, 1, 0).reshape(3136, 128)
    w1t = w1t.astype(jnp.bfloat16)
    w2t = jnp.transpose(fw2).astype(jnp.bfloat16)

    bm = _pick_block(N, 256)
    out = pl.pallas_call(
        _mlp_kernel,
        out_shape=jax.ShapeDtypeStruct((N, 10), jnp.float32),
        grid=(N // bm,),
        in_specs=[
            pl.BlockSpec((bm, 3136), lambda i: (i, 0)),
            pl.BlockSpec((3136, 128), lambda i: (0, 0)),
            pl.BlockSpec((1, 128), lambda i: (0, 0)),
            pl.BlockSpec((128, 10), lambda i: (0, 0)),
            pl.BlockSpec((1, 10), lambda i: (0, 0)),
        ],
        out_specs=pl.BlockSpec((bm, 10), lambda i: (i, 0)),
        compiler_params=pltpu.CompilerParams(dimension_semantics=("parallel",)),
    )(feat, w1t, fb1.reshape(1, 128), w2t, fb2.reshape(1, 10))
    return out


# speedup vs baseline: 1.0237x; 1.0237x over previous
import jax
import jax.numpy as jnp
from jax.experimental import pallas as pl
from jax.experimental.pallas import tpu as pltpu


def _tower_kernel(x1_ref, w1_ref, b1_ref, w2_ref, b2_ref, o_ref, p1_scr,
                  vp2_scr):
    B = x1_ref.shape[0]
    x1 = x1_ref[...].reshape(B * 200, 36)
    c1 = jnp.dot(x1, w1_ref[...], preferred_element_type=jnp.float32)
    c1 = c1.reshape(B, 200, 128)
    hp = jnp.maximum(jnp.maximum(c1[:, :, 0:32], c1[:, :, 32:64]),
                     jnp.maximum(c1[:, :, 64:96], c1[:, :, 96:128]))
    p1 = jnp.maximum(hp + b1_ref[...], 0.0)

    p1_scr[...] = jnp.zeros_like(p1_scr)
    for r in range(14):
        p1_scr[:, pl.ds((r + 1) * 16 + 1, 14), :] = p1[:, r * 14:(r + 1) * 14, :]

    p1f = p1_scr[...].astype(jnp.bfloat16)
    patch = jnp.concatenate(
        [p1f[:, dy * 16 + dx:dy * 16 + dx + 224, :]
         for dy in range(3) for dx in range(3)], axis=-1)
    c2 = jnp.dot(patch.reshape(B * 224, 288), w2_ref[...],
                 preferred_element_type=jnp.float32)
    c2 = c2.reshape(B, 7, 32, 64)
    vp2_scr[...] = jnp.maximum(c2[:, :, 0:14], c2[:, :, 16:30])
    hp2 = jnp.maximum(vp2_scr[:, :, pl.ds(0, 7, stride=2), :],
                      vp2_scr[:, :, pl.ds(1, 7, stride=2), :])
    o_ref[...] = jnp.maximum(hp2 + b2_ref[...], 0.0).astype(o_ref.dtype)


def _mlp_kernel(x_ref, w1_ref, b1_ref, w2_ref, b2_ref, o_ref):
    h = jnp.dot(x_ref[...], w1_ref[...], preferred_element_type=jnp.float32)
    h = jnp.maximum(h + b1_ref[...], 0.0).astype(jnp.bfloat16)
    o_ref[...] = jnp.dot(h, w2_ref[...],
                         preferred_element_type=jnp.float32) + b2_ref[...]


def _pick_block(n, pref):
    b = pref
    while n % b:
        b //= 2
    return max(b, 1)


def kernel(x_nchw, w1_hwio, b1, w2_hwio, b2, fw1, fb1, fw2, fb2):
    N = x_nchw.shape[0]
    x = x_nchw[:, 0]
    xp = jnp.pad(x, ((0, 0), (1, 1), (1, 1)))

    x1 = jnp.stack([xp[:, s + dy:s + dy + 28:2, u + dx:u + dx + 28:2]
                    for s in range(2) for u in range(2)
                    for dy in range(3) for dx in range(3)], axis=-1)
    x1 = jnp.pad(x1.reshape(N, 196, 36), ((0, 0), (0, 4), (0, 0)))
    x1 = x1.astype(jnp.bfloat16)

    w1f = w1_hwio.reshape(9, 32)
    w1p = (jnp.eye(4, dtype=jnp.float32)[:, None, :, None]
           * w1f[None, :, None, :]).reshape(36, 128).astype(jnp.bfloat16)
    w2p = w2_hwio.reshape(9 * 32, 64).astype(jnp.bfloat16)

    bb = _pick_block(N, 64)
    feat = pl.pallas_call(
        _tower_kernel,
        out_shape=jax.ShapeDtypeStruct((N, 7, 7, 64), jnp.bfloat16),
        grid=(N // bb,),
        in_specs=[
            pl.BlockSpec((bb, 200, 36), lambda n: (n, 0, 0)),
            pl.BlockSpec((36, 128), lambda n: (0, 0)),
            pl.BlockSpec((1, 32), lambda n: (0, 0)),
            pl.BlockSpec((288, 64), lambda n: (0, 0)),
            pl.BlockSpec((1, 64), lambda n: (0, 0)),
        ],
        out_specs=pl.BlockSpec((bb, 7, 7, 64), lambda n: (n, 0, 0, 0)),
        scratch_shapes=[pltpu.VMEM((bb, 272, 32), jnp.float32),
                        pltpu.VMEM((bb, 7, 14, 64), jnp.float32)],
        compiler_params=pltpu.CompilerParams(dimension_semantics=("parallel",),
                                            vmem_limit_bytes=100 * 1024 * 1024),
    )(x1, w1p, b1.reshape(1, 32), w2p, b2.reshape(1, 64))

    feat = feat.reshape(N, 3136)

    w1t = fw1.reshape(128, 64, 7, 7).transpose(2, 3, 1, 0).reshape(3136, 128)
    w1t = w1t.astype(jnp.bfloat16)
    w2t = jnp.transpose(fw2).astype(jnp.bfloat16)

    bm = _pick_block(N, 256)
    out = pl.pallas_call(
        _mlp_kernel,
        out_shape=jax.ShapeDtypeStruct((N, 10), jnp.float32),
        grid=(N // bm,),
        in_specs=[
            pl.BlockSpec((bm, 3136), lambda i: (i, 0)),
            pl.BlockSpec((3136, 128), lambda i: (0, 0)),
            pl.BlockSpec((1, 128), lambda i: (0, 0)),
            pl.BlockSpec((128, 10), lambda i: (0, 0)),
            pl.BlockSpec((1, 10), lambda i: (0, 0)),
        ],
        out_specs=pl.BlockSpec((bm, 10), lambda i: (i, 0)),
        compiler_params=pltpu.CompilerParams(dimension_semantics=("parallel",)),
    )(feat, w1t, fb1.reshape(1, 128), w2t, fb2.reshape(1, 10))
    return out


# speedup vs baseline: 1.1168x; 1.0909x over previous
import jax
import jax.numpy as jnp
from jax.experimental import pallas as pl
from jax.experimental.pallas import tpu as pltpu


def _net_kernel(x1_ref, w1_ref, b1_ref, w2_ref, b2_ref, f1_ref, fb1_ref,
                f2_ref, fb2_ref, o_ref, p1_scr, vp2_scr):
    B = x1_ref.shape[0]
    x1 = x1_ref[...].reshape(B * 200, 36)
    c1 = jnp.dot(x1, w1_ref[...], preferred_element_type=jnp.float32)
    c1 = c1.reshape(B, 200, 128)
    hp = jnp.maximum(jnp.maximum(c1[:, :, 0:32], c1[:, :, 32:64]),
                     jnp.maximum(c1[:, :, 64:96], c1[:, :, 96:128]))
    p1 = jnp.maximum(hp + b1_ref[...], 0.0)

    p1_scr[...] = jnp.zeros_like(p1_scr)
    for r in range(14):
        p1_scr[:, pl.ds((r + 1) * 16 + 1, 14), :] = p1[:, r * 14:(r + 1) * 14, :]

    p1f = p1_scr[...].astype(jnp.bfloat16)
    patch = jnp.concatenate(
        [p1f[:, dy * 16 + dx:dy * 16 + dx + 224, :]
         for dy in range(3) for dx in range(3)], axis=-1)
    c2 = jnp.dot(patch.reshape(B * 224, 288), w2_ref[...],
                 preferred_element_type=jnp.float32)
    c2 = c2.reshape(B, 7, 32, 64)
    vp2_scr[...] = jnp.maximum(c2[:, :, 0:14], c2[:, :, 16:30])
    hp2 = jnp.maximum(vp2_scr[:, :, pl.ds(0, 7, stride=2), :],
                      vp2_scr[:, :, pl.ds(1, 7, stride=2), :])
    feat = jnp.maximum(hp2 + b2_ref[...], 0.0).astype(jnp.bfloat16)

    ff = jnp.concatenate([feat[:, r, xx, :] for r in range(7)
                          for xx in range(7)], axis=-1)
    h = jnp.dot(ff, f1_ref[...], preferred_element_type=jnp.float32)
    h = jnp.maximum(h + fb1_ref[...], 0.0).astype(jnp.bfloat16)
    o_ref[...] = jnp.dot(h, f2_ref[...],
                         preferred_element_type=jnp.float32) + fb2_ref[...]


def _pick_block(n, pref):
    b = pref
    while n % b:
        b //= 2
    return max(b, 1)


def kernel(x_nchw, w1_hwio, b1, w2_hwio, b2, fw1, fb1, fw2, fb2):
    N = x_nchw.shape[0]
    x = x_nchw[:, 0]
    xp = jnp.pad(x, ((0, 0), (1, 1), (1, 1)))

    x1 = jnp.stack([xp[:, s + dy:s + dy + 28:2, u + dx:u + dx + 28:2]
                    for s in range(2) for u in range(2)
                    for dy in range(3) for dx in range(3)], axis=-1)
    x1 = jnp.pad(x1.reshape(N, 196, 36), ((0, 0), (0, 4), (0, 0)))
    x1 = x1.astype(jnp.bfloat16)

    w1f = w1_hwio.reshape(9, 32)
    w1p = (jnp.eye(4, dtype=jnp.float32)[:, None, :, None]
           * w1f[None, :, None, :]).reshape(36, 128).astype(jnp.bfloat16)
    w2p = w2_hwio.reshape(9 * 32, 64).astype(jnp.bfloat16)

    w1t = fw1.reshape(128, 64, 7, 7).transpose(2, 3, 1, 0).reshape(3136, 128)
    w1t = w1t.astype(jnp.bfloat16)
    w2t = jnp.transpose(fw2).astype(jnp.bfloat16)

    bb = _pick_block(N, 64)
    out = pl.pallas_call(
        _net_kernel,
        out_shape=jax.ShapeDtypeStruct((N, 10), jnp.float32),
        grid=(N // bb,),
        in_specs=[
            pl.BlockSpec((bb, 200, 36), lambda n: (n, 0, 0)),
            pl.BlockSpec((36, 128), lambda n: (0, 0)),
            pl.BlockSpec((1, 32), lambda n: (0, 0)),
            pl.BlockSpec((288, 64), lambda n: (0, 0)),
            pl.BlockSpec((1, 64), lambda n: (0, 0)),
            pl.BlockSpec((3136, 128), lambda n: (0, 0)),
            pl.BlockSpec((1, 128), lambda n: (0, 0)),
            pl.BlockSpec((128, 10), lambda n: (0, 0)),
            pl.BlockSpec((1, 10), lambda n: (0, 0)),
        ],
        out_specs=pl.BlockSpec((bb, 10), lambda n: (n, 0)),
        scratch_shapes=[pltpu.VMEM((bb, 272, 32), jnp.float32),
                        pltpu.VMEM((bb, 7, 14, 64), jnp.float32)],
        compiler_params=pltpu.CompilerParams(
            dimension_semantics=("parallel",),
            vmem_limit_bytes=100 * 1024 * 1024),
    )(x1, w1p, b1.reshape(1, 32), w2p, b2.reshape(1, 64),
      w1t, fb1.reshape(1, 128), w2t, fb2.reshape(1, 10))
    return out
